# baseline (speedup 1.0000x reference)
import functools

import jax
import jax.numpy as jnp
from jax import lax
from jax.experimental import pallas as pl
from jax.experimental.pallas import tpu as pltpu
from jax.experimental.pallas import tpu_sc as plsc

VOCAB = 1000000
EMBED = 128
B = 4096
L = 200
OUT = 384

NC = 2
NS = 16
NW = NC * NS
RPW = B // NW
LANES = 16
KCH = EMBED // LANES
SPLITS = ((0, 64), (64, 64), (128, 72))
NSPL = len(SPLITS)
BUFROWS = 72
RPT = 4
NBUF = RPT * NSPL
IDXC = 32

_mesh = plsc.VectorSubcoreMesh(core_axis_name="c", subcore_axis_name="s")


@functools.partial(
    pl.kernel,
    out_type=jax.ShapeDtypeStruct((B, EMBED), jnp.float32),
    mesh=_mesh,
    scratch_types=[
        pltpu.VMEM((2, IDXC, L), jnp.int32),
        [pltpu.VMEM((BUFROWS, EMBED), jnp.float32) for _ in range(NBUF)],
        pltpu.VMEM((16, EMBED), jnp.float32),
        [pltpu.SemaphoreType.DMA for _ in range(NBUF)],
        pltpu.SemaphoreType.DMA,
    ],
)
def _bag_kernel(idx_hbm, table_hbm, out_hbm, idx_v, bufs, fbuf, sems, isem):
    wid = lax.axis_index("s") * NC + lax.axis_index("c")
    base = wid * RPW
    NCH = RPW // IDXC
    pltpu.sync_copy(idx_hbm.at[pl.ds(base, IDXC)], idx_v.at[0])
    pltpu.async_copy(idx_hbm.at[pl.ds(base + IDXC, IDXC)], idx_v.at[1], isem)

    def start(row, slot0):
        p = (row // IDXC) % 2
        r = row % IDXC
        for j, (off, ln) in enumerate(SPLITS):
            pltpu.async_copy(
                table_hbm.at[idx_v.at[p, r, pl.ds(off, ln)]],
                bufs[slot0 + j].at[pl.ds(0, ln)], sems[slot0 + j])

    def wait(slot0):
        for j, (off, ln) in enumerate(SPLITS):
            pltpu.make_async_copy(
                table_hbm.at[idx_v.at[0, 0, pl.ds(0, ln)]],
                bufs[slot0 + j].at[pl.ds(0, ln)], sems[slot0 + j]).wait()

    def accum_row(slot0, row):
        def seg(buf, ln, acc):
            def body(i, a):
                aa = list(a)
                for u in range(4):
                    j = 4 * i + u
                    for k in range(KCH):
                        aa[k] = aa[k] + buf[j, pl.ds(k * LANES, LANES)]
                return tuple(aa)

            return lax.fori_loop(0, ln // 4, body, acc)

        acc = tuple(jnp.zeros((LANES,), jnp.float32) for _ in range(KCH))
        for j, (off, ln) in enumerate(SPLITS):
            acc = seg(bufs[slot0 + j], ln, acc)
        for k in range(KCH):
            fbuf[row % 16, pl.ds(k * LANES, LANES)] = acc[k] * jnp.float32(1.0 / L)

    for b in range(RPT):
        start(b, NSPL * b)

    TPC = IDXC // RPT

    def outer(t, carry):
        @pl.when(((t % TPC) == TPC - 1) & (t // TPC + 1 <= NCH - 1))
        def _():
            pltpu.make_async_copy(
                idx_hbm.at[pl.ds(0, IDXC)], idx_v.at[0], isem).wait()

        for b in range(RPT):
            row = RPT * t + b
            wait(NSPL * b)
            accum_row(NSPL * b, row)

            @pl.when(row + RPT < RPW)
            def _():
                start(row + RPT, NSPL * b)

        @pl.when((t & 3) == 3)
        def _():
            pltpu.sync_copy(
                fbuf, out_hbm.at[pl.ds(base + (t // 4) * 16, 16)])

        @pl.when(((t % TPC) == TPC - 1) & (t // TPC + 2 <= NCH - 1))
        def _():
            c2 = t // TPC + 2
            pltpu.async_copy(
                idx_hbm.at[pl.ds(base + c2 * IDXC, IDXC)],
                idx_v.at[(t // TPC) % 2], isem)

        return carry

    lax.fori_loop(0, RPW // RPT, outer, 0)


def _mlp_body(text, url, link, Wu, bu, Wl, bl, W1t, W1u, W1l, bc1, Wc2, bc2, out):
    f32 = jnp.float32
    u = jnp.maximum(jnp.dot(url[...], Wu[...], preferred_element_type=f32) + bu[...], 0.0)
    lv = jnp.maximum(jnp.dot(link[...], Wl[...], preferred_element_type=f32) + bl[...], 0.0)
    h = jnp.dot(text[...], W1t[...], preferred_element_type=f32)
    h = h + jnp.dot(u, W1u[...], preferred_element_type=f32)
    h = h + jnp.dot(lv, W1l[...], preferred_element_type=f32)
    h = jnp.maximum(h + bc1[...], 0.0)
    out[...] = jnp.dot(h, Wc2[...], preferred_element_type=f32) + bc2[...]


_BB = 512


def _mlp(text_vec, url_bits, link_feats, Wu, bu, Wl, bl, W1t, W1u, W1l, bc1, Wc2, bc2):
    n = B // _BB
    row = lambda i: (i, 0)
    rep = lambda i: (0, 0)
    return pl.pallas_call(
        _mlp_body,
        grid=(n,),
        in_specs=[
            pl.BlockSpec((_BB, EMBED), row),
            pl.BlockSpec((_BB, 64), row),
            pl.BlockSpec((_BB, 32), row),
            pl.BlockSpec((64, 64), rep),
            pl.BlockSpec((1, 64), rep),
            pl.BlockSpec((32, 64), rep),
            pl.BlockSpec((1, 64), rep),
            pl.BlockSpec((EMBED, 256), rep),
            pl.BlockSpec((64, 256), rep),
            pl.BlockSpec((64, 256), rep),
            pl.BlockSpec((1, 256), rep),
            pl.BlockSpec((256, OUT), rep),
            pl.BlockSpec((1, OUT), rep),
        ],
        out_specs=pl.BlockSpec((_BB, OUT), row),
        out_shape=jax.ShapeDtypeStruct((B, OUT), jnp.float32),
    )(text_vec, url_bits, link_feats, Wu, bu, Wl, bl, W1t, W1u, W1l, bc1, Wc2, bc2)


def kernel(text_indices, url_bits, link_feats, text_table, Wu, bu, Wl, bl, Wc1, bc1, Wc2, bc2):
    idx = text_indices.astype(jnp.int32)
    text_vec = _bag_kernel(idx, text_table)
    W1t = Wc1[:EMBED]
    W1u = Wc1[EMBED:EMBED + 64]
    W1l = Wc1[EMBED + 64:]
    return _mlp(
        text_vec, url_bits, link_feats,
        Wu, bu.reshape(1, 64), Wl, bl.reshape(1, 64),
        W1t, W1u, W1l, bc1.reshape(1, 256), Wc2, bc2.reshape(1, OUT))


# speedup vs baseline: 1.0974x; 1.0974x over previous
import functools

import jax
import jax.numpy as jnp
from jax import lax
from jax.experimental import pallas as pl
from jax.experimental.pallas import tpu as pltpu
from jax.experimental.pallas import tpu_sc as plsc

VOCAB = 1000000
EMBED = 128
B = 4096
L = 200
OUT = 384

NC = 2
NS = 16
NW = NC * NS
RPW = B // NW
LANES = 16
KCH = EMBED // LANES
C0, C1 = 128, L - 128
IDXC = 32

_mesh = plsc.VectorSubcoreMesh(core_axis_name="c", subcore_axis_name="s")


@functools.partial(
    pl.kernel,
    out_type=jax.ShapeDtypeStruct((B, EMBED), jnp.float32),
    mesh=_mesh,
    scratch_types=[
        pltpu.VMEM((2, IDXC, L), jnp.int32),
        pltpu.VMEM((L, EMBED), jnp.float32),
        pltpu.VMEM((L, EMBED), jnp.float32),
        pltpu.VMEM((L, EMBED), jnp.float32),
        pltpu.VMEM((L, EMBED), jnp.float32),
        pltpu.VMEM((16, EMBED), jnp.float32),
        pltpu.SemaphoreType.DMA,
        pltpu.SemaphoreType.DMA,
        pltpu.SemaphoreType.DMA,
        pltpu.SemaphoreType.DMA,
        pltpu.SemaphoreType.DMA,
    ],
)
def _bag_kernel(idx_hbm, table_hbm, out_hbm, idx_v, buf0, buf1, buf2, buf3,
                fbuf, sem0, sem1, sem2, sem3, isem):
    wid = lax.axis_index("s") * NC + lax.axis_index("c")
    base = wid * RPW
    NCH = RPW // IDXC
    pltpu.sync_copy(idx_hbm.at[pl.ds(base, IDXC)], idx_v.at[0])
    pltpu.async_copy(idx_hbm.at[pl.ds(base + IDXC, IDXC)], idx_v.at[1], isem)

    bufs = (buf0, buf1, buf2, buf3)
    sems = (sem0, sem1, sem2, sem3)
    NBUF = 4

    def start(row, buf, sem):
        p = (row // IDXC) % 2
        r = row % IDXC
        pltpu.async_copy(
            table_hbm.at[idx_v.at[p, r, pl.ds(0, C0)]], buf.at[pl.ds(0, C0)], sem)
        pltpu.async_copy(
            table_hbm.at[idx_v.at[p, r, pl.ds(C0, C1)]], buf.at[pl.ds(C0, C1)], sem)

    def wait(buf, sem):
        pltpu.make_async_copy(
            table_hbm.at[idx_v.at[0, 0, pl.ds(0, C0)]], buf.at[pl.ds(0, C0)], sem).wait()
        pltpu.make_async_copy(
            table_hbm.at[idx_v.at[0, 0, pl.ds(C0, C1)]], buf.at[pl.ds(C0, C1)], sem).wait()

    def accum_row(buf, row):
        def body(i, acc):
            a = list(acc)
            for u in range(4):
                j = 4 * i + u
                for k in range(KCH):
                    a[k] = a[k] + buf[j, pl.ds(k * LANES, LANES)]
            return tuple(a)

        acc = lax.fori_loop(
            0, L // 4, body,
            tuple(jnp.zeros((LANES,), jnp.float32) for _ in range(KCH)))
        for k in range(KCH):
            fbuf[row % 16, pl.ds(k * LANES, LANES)] = acc[k] * jnp.float32(1.0 / L)

    for b in range(NBUF):
        start(b, bufs[b], sems[b])

    TPC = IDXC // NBUF

    def outer(t, carry):
        @pl.when(((t % TPC) == TPC - 1) & (t // TPC + 1 <= NCH - 1))
        def _():
            pltpu.make_async_copy(
                idx_hbm.at[pl.ds(0, IDXC)], idx_v.at[0], isem).wait()

        for b in range(NBUF):
            row = NBUF * t + b
            wait(bufs[b], sems[b])
            accum_row(bufs[b], row)

            @pl.when(row + NBUF < RPW)
            def _():
                start(row + NBUF, bufs[b], sems[b])

        @pl.when((t & 3) == 3)
        def _():
            pltpu.sync_copy(
                fbuf, out_hbm.at[pl.ds(base + (t // 4) * 16, 16)])

        @pl.when(((t % TPC) == TPC - 1) & (t // TPC + 2 <= NCH - 1))
        def _():
            c2 = t // TPC + 2
            pltpu.async_copy(
                idx_hbm.at[pl.ds(base + c2 * IDXC, IDXC)],
                idx_v.at[(t // TPC) % 2], isem)

        return carry

    lax.fori_loop(0, RPW // NBUF, outer, 0)


def _mlp_body(text, url, link, Wu, bu, Wl, bl, W1t, W1u, W1l, bc1, Wc2, bc2, out):
    f32 = jnp.float32
    u = jnp.maximum(jnp.dot(url[...], Wu[...], preferred_element_type=f32) + bu[...], 0.0)
    lv = jnp.maximum(jnp.dot(link[...], Wl[...], preferred_element_type=f32) + bl[...], 0.0)
    h = jnp.dot(text[...], W1t[...], preferred_element_type=f32)
    h = h + jnp.dot(u, W1u[...], preferred_element_type=f32)
    h = h + jnp.dot(lv, W1l[...], preferred_element_type=f32)
    h = jnp.maximum(h + bc1[...], 0.0)
    out[...] = jnp.dot(h, Wc2[...], preferred_element_type=f32) + bc2[...]


_BB = 512


def _mlp(text_vec, url_bits, link_feats, Wu, bu, Wl, bl, W1t, W1u, W1l, bc1, Wc2, bc2):
    n = B // _BB
    row = lambda i: (i, 0)
    rep = lambda i: (0, 0)
    return pl.pallas_call(
        _mlp_body,
        grid=(n,),
        in_specs=[
            pl.BlockSpec((_BB, EMBED), row),
            pl.BlockSpec((_BB, 64), row),
            pl.BlockSpec((_BB, 32), row),
            pl.BlockSpec((64, 64), rep),
            pl.BlockSpec((1, 64), rep),
            pl.BlockSpec((32, 64), rep),
            pl.BlockSpec((1, 64), rep),
            pl.BlockSpec((EMBED, 256), rep),
            pl.BlockSpec((64, 256), rep),
            pl.BlockSpec((64, 256), rep),
            pl.BlockSpec((1, 256), rep),
            pl.BlockSpec((256, OUT), rep),
            pl.BlockSpec((1, OUT), rep),
        ],
        out_specs=pl.BlockSpec((_BB, OUT), row),
        out_shape=jax.ShapeDtypeStruct((B, OUT), jnp.float32),
    )(text_vec, url_bits, link_feats, Wu, bu, Wl, bl, W1t, W1u, W1l, bc1, Wc2, bc2)


def kernel(text_indices, url_bits, link_feats, text_table, Wu, bu, Wl, bl, Wc1, bc1, Wc2, bc2):
    idx = text_indices.astype(jnp.int32)
    text_vec = _bag_kernel(idx, text_table)
    W1t = Wc1[:EMBED]
    W1u = Wc1[EMBED:EMBED + 64]
    W1l = Wc1[EMBED + 64:]
    return _mlp(
        text_vec, url_bits, link_feats,
        Wu, bu.reshape(1, 64), Wl, bl.reshape(1, 64),
        W1t, W1u, W1l, bc1.reshape(1, 256), Wc2, bc2.reshape(1, OUT))


# speedup vs baseline: 1.1121x; 1.0134x over previous
import functools

import jax
import jax.numpy as jnp
from jax import lax
from jax.experimental import pallas as pl
from jax.experimental.pallas import tpu as pltpu
from jax.experimental.pallas import tpu_sc as plsc

VOCAB = 1000000
EMBED = 128
B = 4096
L = 200
OUT = 384

NC = 2
NS = 16
NW = NC * NS
RPW = B // NW
LANES = 16
KCH = EMBED // LANES
C0, C1 = 128, L - 128
IDXC = 32

_mesh = plsc.VectorSubcoreMesh(core_axis_name="c", subcore_axis_name="s")


@functools.partial(
    pl.kernel,
    out_type=jax.ShapeDtypeStruct((B, EMBED), jnp.float32),
    mesh=_mesh,
    scratch_types=[
        pltpu.VMEM((2, IDXC, L), jnp.int32),
        pltpu.VMEM((L, EMBED), jnp.float32),
        pltpu.VMEM((L, EMBED), jnp.float32),
        pltpu.VMEM((L, EMBED), jnp.float32),
        pltpu.VMEM((L, EMBED), jnp.float32),
        pltpu.VMEM((16, EMBED), jnp.float32),
        pltpu.SemaphoreType.DMA,
        pltpu.SemaphoreType.DMA,
        pltpu.SemaphoreType.DMA,
        pltpu.SemaphoreType.DMA,
        pltpu.SemaphoreType.DMA,
    ],
)
def _bag_kernel(idx_hbm, table_hbm, out_hbm, idx_v, buf0, buf1, buf2, buf3,
                fbuf, sem0, sem1, sem2, sem3, isem):
    wid = lax.axis_index("s") * NC + lax.axis_index("c")
    base = wid * RPW
    NCH = RPW // IDXC
    pltpu.sync_copy(idx_hbm.at[pl.ds(base, IDXC)], idx_v.at[0])
    pltpu.async_copy(idx_hbm.at[pl.ds(base + IDXC, IDXC)], idx_v.at[1], isem)

    bufs = (buf0, buf1, buf2, buf3)
    sems = (sem0, sem1, sem2, sem3)
    NBUF = 4

    def start(row, buf, sem):
        p = (row // IDXC) % 2
        r = row % IDXC
        pltpu.async_copy(
            table_hbm.at[idx_v.at[p, r, pl.ds(0, C0)]], buf.at[pl.ds(0, C0)], sem)
        pltpu.async_copy(
            table_hbm.at[idx_v.at[p, r, pl.ds(C0, C1)]], buf.at[pl.ds(C0, C1)], sem)

    def wait(buf, sem):
        pltpu.make_async_copy(
            table_hbm.at[idx_v.at[0, 0, pl.ds(0, C0)]], buf.at[pl.ds(0, C0)], sem).wait()
        pltpu.make_async_copy(
            table_hbm.at[idx_v.at[0, 0, pl.ds(C0, C1)]], buf.at[pl.ds(C0, C1)], sem).wait()

    def accum_row(buf, row):
        def body(i, acc):
            a = list(acc)
            for u in range(4):
                j = 4 * i + u
                for k in range(KCH):
                    a[k] = a[k] + buf[j, pl.ds(k * LANES, LANES)]
            return tuple(a)

        acc = lax.fori_loop(
            0, L // 4, body,
            tuple(jnp.zeros((LANES,), jnp.float32) for _ in range(KCH)))
        for k in range(KCH):
            fbuf[row % 16, pl.ds(k * LANES, LANES)] = acc[k] * jnp.float32(1.0 / L)

    for b in range(NBUF):
        start(b, bufs[b], sems[b])

    TPC = IDXC // NBUF

    def outer(t, carry):
        @pl.when(((t % TPC) == TPC - 1) & (t // TPC + 1 <= NCH - 1))
        def _():
            pltpu.make_async_copy(
                idx_hbm.at[pl.ds(0, IDXC)], idx_v.at[0], isem).wait()

        for b in range(NBUF):
            row = NBUF * t + b
            wait(bufs[b], sems[b])
            accum_row(bufs[b], row)

            @pl.when(row + NBUF < RPW)
            def _():
                start(row + NBUF, bufs[b], sems[b])

        @pl.when((t & 3) == 3)
        def _():
            pltpu.sync_copy(
                fbuf, out_hbm.at[pl.ds(base + (t // 4) * 16, 16)])

        @pl.when(((t % TPC) == TPC - 1) & (t // TPC + 2 <= NCH - 1))
        def _():
            c2 = t // TPC + 2
            pltpu.async_copy(
                idx_hbm.at[pl.ds(base + c2 * IDXC, IDXC)],
                idx_v.at[(t // TPC) % 2], isem)

        return carry

    lax.fori_loop(0, RPW // NBUF, outer, 0)


def _mlp_body(text, url, link, Wu, bu, Wl, bl, W1t, W1u, W1l, bc1, Wc2, bc2, out):
    f32 = jnp.float32
    u = jnp.maximum(jnp.dot(url[...], Wu[...], preferred_element_type=f32) + bu[...], 0.0)
    lv = jnp.maximum(jnp.dot(link[...], Wl[...], preferred_element_type=f32) + bl[...], 0.0)
    h = jnp.dot(text[...], W1t[...], preferred_element_type=f32)
    h = h + jnp.dot(u, W1u[...], preferred_element_type=f32)
    h = h + jnp.dot(lv, W1l[...], preferred_element_type=f32)
    h = jnp.maximum(h + bc1[...], 0.0)
    out[...] = jnp.dot(h, Wc2[...], preferred_element_type=f32) + bc2[...]


_BB = 1024


def _mlp(text_vec, url_bits, link_feats, Wu, bu, Wl, bl, W1t, W1u, W1l, bc1, Wc2, bc2):
    n = B // _BB
    row = lambda i: (i, 0)
    rep = lambda i: (0, 0)
    return pl.pallas_call(
        _mlp_body,
        grid=(n,),
        in_specs=[
            pl.BlockSpec((_BB, EMBED), row),
            pl.BlockSpec((_BB, 64), row),
            pl.BlockSpec((_BB, 32), row),
            pl.BlockSpec((64, 64), rep),
            pl.BlockSpec((1, 64), rep),
            pl.BlockSpec((32, 64), rep),
            pl.BlockSpec((1, 64), rep),
            pl.BlockSpec((EMBED, 256), rep),
            pl.BlockSpec((64, 256), rep),
            pl.BlockSpec((64, 256), rep),
            pl.BlockSpec((1, 256), rep),
            pl.BlockSpec((256, OUT), rep),
            pl.BlockSpec((1, OUT), rep),
        ],
        out_specs=pl.BlockSpec((_BB, OUT), row),
        out_shape=jax.ShapeDtypeStruct((B, OUT), jnp.float32),
    )(text_vec, url_bits, link_feats, Wu, bu, Wl, bl, W1t, W1u, W1l, bc1, Wc2, bc2)


def kernel(text_indices, url_bits, link_feats, text_table, Wu, bu, Wl, bl, Wc1, bc1, Wc2, bc2):
    idx = text_indices.astype(jnp.int32)
    text_vec = _bag_kernel(idx, text_table)
    W1t = Wc1[:EMBED]
    W1u = Wc1[EMBED:EMBED + 64]
    W1l = Wc1[EMBED + 64:]
    return _mlp(
        text_vec, url_bits, link_feats,
        Wu, bu.reshape(1, 64), Wl, bl.reshape(1, 64),
        W1t, W1u, W1l, bc1.reshape(1, 256), Wc2, bc2.reshape(1, OUT))


# speedup vs baseline: 1.1171x; 1.0045x over previous
import functools

import jax
import jax.numpy as jnp
from jax import lax
from jax.experimental import pallas as pl
from jax.experimental.pallas import tpu as pltpu
from jax.experimental.pallas import tpu_sc as plsc

VOCAB = 1000000
EMBED = 128
B = 4096
L = 200
OUT = 384

NC = 2
NS = 16
NW = NC * NS
RPW = B // NW
LANES = 16
KCH = EMBED // LANES
C0, C1 = 128, L - 128
IDXC = 32

_mesh = plsc.VectorSubcoreMesh(core_axis_name="c", subcore_axis_name="s")


@functools.partial(
    pl.kernel,
    out_type=jax.ShapeDtypeStruct((B, EMBED), jnp.float32),
    mesh=_mesh,
    scratch_types=[
        pltpu.VMEM((2, IDXC, L), jnp.int32),
        pltpu.VMEM((L, EMBED), jnp.float32),
        pltpu.VMEM((L, EMBED), jnp.float32),
        pltpu.VMEM((L, EMBED), jnp.float32),
        pltpu.VMEM((L, EMBED), jnp.float32),
        pltpu.VMEM((16, EMBED), jnp.float32),
        pltpu.SemaphoreType.DMA,
        pltpu.SemaphoreType.DMA,
        pltpu.SemaphoreType.DMA,
        pltpu.SemaphoreType.DMA,
        pltpu.SemaphoreType.DMA,
    ],
)
def _bag_kernel(idx_hbm, table_hbm, out_hbm, idx_v, buf0, buf1, buf2, buf3,
                fbuf, sem0, sem1, sem2, sem3, isem):
    wid = lax.axis_index("s") * NC + lax.axis_index("c")
    base = wid * RPW
    NCH = RPW // IDXC
    pltpu.sync_copy(idx_hbm.at[pl.ds(base, IDXC)], idx_v.at[0])
    pltpu.async_copy(idx_hbm.at[pl.ds(base + IDXC, IDXC)], idx_v.at[1], isem)

    bufs = (buf0, buf1, buf2, buf3)
    sems = (sem0, sem1, sem2, sem3)
    NBUF = 4

    def start(row, buf, sem):
        p = (row // IDXC) % 2
        r = row % IDXC
        pltpu.async_copy(
            table_hbm.at[idx_v.at[p, r, pl.ds(0, C0)]], buf.at[pl.ds(0, C0)], sem)
        pltpu.async_copy(
            table_hbm.at[idx_v.at[p, r, pl.ds(C0, C1)]], buf.at[pl.ds(C0, C1)], sem)

    def wait(buf, sem):
        pltpu.make_async_copy(
            table_hbm.at[idx_v.at[0, 0, pl.ds(0, C0)]], buf.at[pl.ds(0, C0)], sem).wait()
        pltpu.make_async_copy(
            table_hbm.at[idx_v.at[0, 0, pl.ds(C0, C1)]], buf.at[pl.ds(C0, C1)], sem).wait()

    def accum_row(buf, row):
        def body(i, acc):
            a = list(acc)
            for u in range(4):
                j = 4 * i + u
                for k in range(KCH):
                    a[k] = a[k] + buf[j, pl.ds(k * LANES, LANES)]
            return tuple(a)

        acc = lax.fori_loop(
            0, L // 4, body,
            tuple(jnp.zeros((LANES,), jnp.float32) for _ in range(KCH)))
        for k in range(KCH):
            fbuf[row % 16, pl.ds(k * LANES, LANES)] = acc[k] * jnp.float32(1.0 / L)

    for b in range(NBUF):
        start(b, bufs[b], sems[b])

    TPC = IDXC // NBUF

    def outer(t, carry):
        @pl.when(((t % TPC) == TPC - 1) & (t // TPC + 1 <= NCH - 1))
        def _():
            pltpu.make_async_copy(
                idx_hbm.at[pl.ds(0, IDXC)], idx_v.at[0], isem).wait()

        for b in range(NBUF):
            row = NBUF * t + b
            wait(bufs[b], sems[b])
            accum_row(bufs[b], row)

            @pl.when(row + NBUF < RPW)
            def _():
                start(row + NBUF, bufs[b], sems[b])

        @pl.when((t & 3) == 3)
        def _():
            pltpu.sync_copy(
                fbuf, out_hbm.at[pl.ds(base + (t // 4) * 16, 16)])

        @pl.when(((t % TPC) == TPC - 1) & (t // TPC + 2 <= NCH - 1))
        def _():
            c2 = t // TPC + 2
            pltpu.async_copy(
                idx_hbm.at[pl.ds(base + c2 * IDXC, IDXC)],
                idx_v.at[(t // TPC) % 2], isem)

        return carry

    lax.fori_loop(0, RPW // NBUF, outer, 0)


def _mlp_body(text, url, link, Wu, bu, Wl, bl, W1t, W1u, W1l, bc1, Wc2, bc2, out):
    f32 = jnp.float32
    u = jnp.maximum(jnp.dot(url[...], Wu[...], preferred_element_type=f32) + bu[...], 0.0)
    lv = jnp.maximum(jnp.dot(link[...], Wl[...], preferred_element_type=f32) + bl[...], 0.0)
    h = jnp.dot(text[...], W1t[...], preferred_element_type=f32)
    h = h + jnp.dot(u, W1u[...], preferred_element_type=f32)
    h = h + jnp.dot(lv, W1l[...], preferred_element_type=f32)
    h = jnp.maximum(h + bc1[...], 0.0)
    out[...] = jnp.dot(h, Wc2[...], preferred_element_type=f32) + bc2[...]


_BB = 2048


def _mlp(text_vec, url_bits, link_feats, Wu, bu, Wl, bl, W1t, W1u, W1l, bc1, Wc2, bc2):
    n = B // _BB
    row = lambda i: (i, 0)
    rep = lambda i: (0, 0)
    return pl.pallas_call(
        _mlp_body,
        grid=(n,),
        in_specs=[
            pl.BlockSpec((_BB, EMBED), row),
            pl.BlockSpec((_BB, 64), row),
            pl.BlockSpec((_BB, 32), row),
            pl.BlockSpec((64, 64), rep),
            pl.BlockSpec((1, 64), rep),
            pl.BlockSpec((32, 64), rep),
            pl.BlockSpec((1, 64), rep),
            pl.BlockSpec((EMBED, 256), rep),
            pl.BlockSpec((64, 256), rep),
            pl.BlockSpec((64, 256), rep),
            pl.BlockSpec((1, 256), rep),
            pl.BlockSpec((256, OUT), rep),
            pl.BlockSpec((1, OUT), rep),
        ],
        out_specs=pl.BlockSpec((_BB, OUT), row),
        out_shape=jax.ShapeDtypeStruct((B, OUT), jnp.float32),
    )(text_vec, url_bits, link_feats, Wu, bu, Wl, bl, W1t, W1u, W1l, bc1, Wc2, bc2)


def kernel(text_indices, url_bits, link_feats, text_table, Wu, bu, Wl, bl, Wc1, bc1, Wc2, bc2):
    idx = text_indices.astype(jnp.int32)
    text_vec = _bag_kernel(idx, text_table)
    W1t = Wc1[:EMBED]
    W1u = Wc1[EMBED:EMBED + 64]
    W1l = Wc1[EMBED + 64:]
    return _mlp(
        text_vec, url_bits, link_feats,
        Wu, bu.reshape(1, 64), Wl, bl.reshape(1, 64),
        W1t, W1u, W1l, bc1.reshape(1, 256), Wc2, bc2.reshape(1, OUT))
